# baseline (speedup 1.0000x reference)
import jax
import jax.numpy as jnp
from jax import lax
from jax.experimental import pallas as pl
from jax.experimental.pallas import tpu as pltpu
from jax.experimental.pallas import tpu_sc as plsc

T = 1000
TPAD = 1024
LATENT_DIM = 128
BATCH = 16384

B_SC = 4096
B_TC = BATCH - B_SC
BLK = 1024

_info = plsc.get_sparse_core_info()
_NC, _NS = _info.num_cores, _info.num_subcores
_NW = _NC * _NS
_CHUNK = 128
_ROWS_PER_W = B_SC // _NW
_CHUNKS_PER_W = _ROWS_PER_W // _CHUNK


def _sc_body(t_hbm, table_hbm, out_hbm, idx_v, rows_v, sem):
    wid = lax.axis_index("s") * _NC + lax.axis_index("c")
    base = wid * _ROWS_PER_W
    pltpu.sync_copy(t_hbm.at[pl.ds(base, _ROWS_PER_W)], idx_v)
    descs = []
    for j in range(_CHUNKS_PER_W):
        descs.append(
            pltpu.async_copy(
                table_hbm.at[idx_v.at[pl.ds(j * _CHUNK, _CHUNK)]],
                rows_v.at[pl.ds(j * _CHUNK, _CHUNK)],
                sem,
            )
        )
    for d in descs:
        d.wait()
    pltpu.sync_copy(rows_v, out_hbm.at[pl.ds(base, _ROWS_PER_W)])


def _sc_gather(t, table):
    mesh = plsc.VectorSubcoreMesh(core_axis_name="c", subcore_axis_name="s")
    return pl.kernel(
        _sc_body,
        out_type=jax.ShapeDtypeStruct((B_SC, LATENT_DIM), jnp.float32),
        mesh=mesh,
        scratch_types=[
            pltpu.VMEM((_ROWS_PER_W,), jnp.int32),
            pltpu.VMEM((_ROWS_PER_W, LATENT_DIM), jnp.float32),
            pltpu.SemaphoreType.DMA,
        ],
    )(t, table)


def _tc_body(t_ref, tab_hi_ref, tab_lo_ref, out_ref):
    t_blk = t_ref[0, 0]
    ks = jax.lax.broadcasted_iota(jnp.int32, (BLK, TPAD), 1)
    oh = (t_blk[:, None] == ks).astype(jnp.bfloat16)
    acc = jnp.dot(oh, tab_hi_ref[...], preferred_element_type=jnp.float32)
    acc = acc + jnp.dot(oh, tab_lo_ref[...], preferred_element_type=jnp.float32)
    out_ref[...] = acc


def _tc_matmul(t, table):
    tab_hi = table.astype(jnp.bfloat16)
    tab_lo = (table - tab_hi.astype(jnp.float32)).astype(jnp.bfloat16)
    pad = [(0, TPAD - T), (0, 0)]
    tab_hi = jnp.pad(tab_hi, pad)
    tab_lo = jnp.pad(tab_lo, pad)
    t_3d = t.reshape(BATCH // BLK, 1, BLK)
    off = B_SC // BLK
    return pl.pallas_call(
        _tc_body,
        grid=(B_TC // BLK,),
        in_specs=[
            pl.BlockSpec((1, 1, BLK), lambda i: (i + off, 0, 0)),
            pl.BlockSpec((TPAD, LATENT_DIM), lambda i: (0, 0)),
            pl.BlockSpec((TPAD, LATENT_DIM), lambda i: (0, 0)),
        ],
        out_specs=pl.BlockSpec((BLK, LATENT_DIM), lambda i: (i, 0)),
        out_shape=jax.ShapeDtypeStruct((B_TC, LATENT_DIM), jnp.float32),
    )(t_3d, tab_hi, tab_lo)


def kernel(t, table):
    t = t.astype(jnp.int32)
    sc_out = _sc_gather(t, table)
    tc_out = _tc_matmul(t, table)
    return jnp.concatenate([sc_out, tc_out], axis=0)


# speedup vs baseline: 1.3510x; 1.3510x over previous
import jax
import jax.numpy as jnp
from jax import lax
from jax.experimental import pallas as pl
from jax.experimental.pallas import tpu as pltpu
from jax.experimental.pallas import tpu_sc as plsc

T = 1000
LATENT_DIM = 128
BATCH = 16384

B_SC = 12288
B_TC = BATCH - B_SC
BLK = 1024

_info = plsc.get_sparse_core_info()
_NC, _NS = _info.num_cores, _info.num_subcores
_NW = _NC * _NS
_CHUNK = 128
_ROWS_PER_W = B_SC // _NW
_CHUNKS_PER_W = _ROWS_PER_W // _CHUNK


def _sc_body(t_hbm, table_hbm, out_hbm, idx_v, rows_v, sem):
    wid = lax.axis_index("s") * _NC + lax.axis_index("c")
    base = wid * _ROWS_PER_W
    pltpu.sync_copy(t_hbm.at[pl.ds(base, _ROWS_PER_W)], idx_v)
    descs = []
    for j in range(_CHUNKS_PER_W):
        descs.append(
            pltpu.async_copy(
                table_hbm.at[idx_v.at[pl.ds(j * _CHUNK, _CHUNK)]],
                rows_v.at[pl.ds(j * _CHUNK, _CHUNK)],
                sem,
            )
        )
    for d in descs:
        d.wait()
    pltpu.sync_copy(rows_v, out_hbm.at[pl.ds(base, _ROWS_PER_W)])


def _sc_gather(t, table):
    mesh = plsc.VectorSubcoreMesh(core_axis_name="c", subcore_axis_name="s")
    return pl.kernel(
        _sc_body,
        out_type=jax.ShapeDtypeStruct((BATCH, LATENT_DIM), jnp.float32),
        mesh=mesh,
        scratch_types=[
            pltpu.VMEM((_ROWS_PER_W,), jnp.int32),
            pltpu.VMEM((_ROWS_PER_W, LATENT_DIM), jnp.float32),
            pltpu.SemaphoreType.DMA,
        ],
    )(t, table)


def _tc_body(t_ref, tab_hi_ref, tab_lo_ref, out_ref):
    t_blk = t_ref[0, 0]
    ks = jax.lax.broadcasted_iota(jnp.int32, (BLK, T), 1)
    oh = (t_blk[:, None] == ks).astype(jnp.bfloat16)
    acc = jnp.dot(oh, tab_hi_ref[...], preferred_element_type=jnp.float32)
    acc = acc + jnp.dot(oh, tab_lo_ref[...], preferred_element_type=jnp.float32)
    out_ref[...] = acc


def _tc_matmul(t, table):
    tab_hi = table.astype(jnp.bfloat16)
    tab_lo = (table - tab_hi.astype(jnp.float32)).astype(jnp.bfloat16)
    t_3d = t.reshape(BATCH // BLK, 1, BLK)
    off = B_SC // BLK
    return pl.pallas_call(
        _tc_body,
        grid=(B_TC // BLK,),
        in_specs=[
            pl.BlockSpec((1, 1, BLK), lambda i: (i + off, 0, 0)),
            pl.BlockSpec((T, LATENT_DIM), lambda i: (0, 0)),
            pl.BlockSpec((T, LATENT_DIM), lambda i: (0, 0)),
        ],
        out_specs=pl.BlockSpec((BLK, LATENT_DIM), lambda i: (i, 0)),
        out_shape=jax.ShapeDtypeStruct((B_TC, LATENT_DIM), jnp.float32),
    )(t_3d, tab_hi, tab_lo)


def kernel(t, table):
    t = t.astype(jnp.int32)
    sc_full = _sc_gather(t, table)
    tc_out = _tc_matmul(t, table)
    return lax.dynamic_update_slice(sc_full, tc_out, (B_SC, 0))


# speedup vs baseline: 1.3928x; 1.0310x over previous
import jax
import jax.numpy as jnp
from jax import lax
from jax.experimental import pallas as pl
from jax.experimental.pallas import tpu as pltpu
from jax.experimental.pallas import tpu_sc as plsc

T = 1000
LATENT_DIM = 128
BATCH = 16384

_info = plsc.get_sparse_core_info()
_NC, _NS = _info.num_cores, _info.num_subcores
_NW = _NC * _NS
_CHUNK = 512
_ROWS_PER_W = BATCH // _NW
_CHUNKS_PER_W = _ROWS_PER_W // _CHUNK


def _gather_body(t_hbm, table_hbm, out_hbm, idx_v, rows_v, sem):
    wid = lax.axis_index("s") * _NC + lax.axis_index("c")
    base = wid * _ROWS_PER_W
    pltpu.sync_copy(t_hbm.at[pl.ds(base, _ROWS_PER_W)], idx_v)
    descs = []
    for j in range(_CHUNKS_PER_W):
        descs.append(
            pltpu.async_copy(
                table_hbm.at[idx_v.at[pl.ds(j * _CHUNK, _CHUNK)]],
                rows_v.at[pl.ds(j * _CHUNK, _CHUNK)],
                sem,
            )
        )
    for d in descs:
        d.wait()
    pltpu.sync_copy(rows_v, out_hbm.at[pl.ds(base, _ROWS_PER_W)])


def kernel(t, table):
    mesh = plsc.VectorSubcoreMesh(core_axis_name="c", subcore_axis_name="s")
    return pl.kernel(
        _gather_body,
        out_type=jax.ShapeDtypeStruct((BATCH, LATENT_DIM), jnp.float32),
        mesh=mesh,
        scratch_types=[
            pltpu.VMEM((_ROWS_PER_W,), jnp.int32),
            pltpu.VMEM((_ROWS_PER_W, LATENT_DIM), jnp.float32),
            pltpu.SemaphoreType.DMA,
        ],
    )(t, table)


# speedup vs baseline: 1.4031x; 1.0073x over previous
import jax
import jax.numpy as jnp
from jax import lax
from jax.experimental import pallas as pl
from jax.experimental.pallas import tpu as pltpu
from jax.experimental.pallas import tpu_sc as plsc

T = 1000
LATENT_DIM = 128
BATCH = 16384

_info = plsc.get_sparse_core_info()
_NC, _NS = _info.num_cores, _info.num_subcores
_NW = _NC * _NS
_CHUNK = 128
_ROWS_PER_W = BATCH // _NW
_CHUNKS_PER_W = _ROWS_PER_W // _CHUNK


def _gather_body(t_hbm, table_hbm, out_hbm, idx_v, rows_v, sem):
    wid = lax.axis_index("s") * _NC + lax.axis_index("c")
    base = wid * _ROWS_PER_W
    pltpu.sync_copy(t_hbm.at[pl.ds(base, _ROWS_PER_W)], idx_v)
    descs = []
    for j in range(_CHUNKS_PER_W):
        descs.append(
            pltpu.async_copy(
                table_hbm.at[idx_v.at[pl.ds(j * _CHUNK, _CHUNK)]],
                rows_v.at[pl.ds(j * _CHUNK, _CHUNK)],
                sem,
            )
        )
    for d in descs:
        d.wait()
    pltpu.sync_copy(rows_v, out_hbm.at[pl.ds(base, _ROWS_PER_W)])


def kernel(t, table):
    mesh = plsc.VectorSubcoreMesh(core_axis_name="c", subcore_axis_name="s")
    return pl.kernel(
        _gather_body,
        out_type=jax.ShapeDtypeStruct((BATCH, LATENT_DIM), jnp.float32),
        mesh=mesh,
        scratch_types=[
            pltpu.VMEM((_ROWS_PER_W,), jnp.int32),
            pltpu.VMEM((_ROWS_PER_W, LATENT_DIM), jnp.float32),
            pltpu.SemaphoreType.DMA,
        ],
    )(t, table)
